# Initial kernel scaffold; baseline (speedup 1.0000x reference)
import jax
import jax.numpy as jnp
from jax.experimental import pallas as pl


def kernel(prob_A):
    raise NotImplementedError("write your pallas kernel here")


# speedup vs baseline: 179.1663x; 179.1663x over previous
import functools

import jax
import jax.numpy as jnp
import numpy as np
from jax.experimental import pallas as pl
from jax.experimental.pallas import tpu as pltpu

_BLK = 512

_ROT_A = (13, 15, 26, 6)
_ROT_B = (17, 29, 16, 24)


def _rounds(x0, x1, rots):
    for r in rots:
        x0 = x0 + x1
        x1 = (x1 << r) | (x1 >> (32 - r))
        x1 = x1 ^ x0
    return x0, x1


def _threefry_bits(idx):
    ks0 = jnp.uint32(0)
    ks1 = jnp.uint32(1)
    ks2 = jnp.uint32(0x1BD11BDA ^ 0 ^ 1)
    x0 = jnp.full(idx.shape, ks0, jnp.uint32)
    x1 = idx + ks1
    x0, x1 = _rounds(x0, x1, _ROT_A)
    x0 = x0 + ks1
    x1 = x1 + ks2 + jnp.uint32(1)
    x0, x1 = _rounds(x0, x1, _ROT_B)
    x0 = x0 + ks2
    x1 = x1 + ks0 + jnp.uint32(2)
    x0, x1 = _rounds(x0, x1, _ROT_A)
    x0 = x0 + ks0
    x1 = x1 + ks1 + jnp.uint32(3)
    x0, x1 = _rounds(x0, x1, _ROT_B)
    x0 = x0 + ks1
    x1 = x1 + ks2 + jnp.uint32(4)
    x0, x1 = _rounds(x0, x1, _ROT_A)
    x0 = x0 + ks2
    x1 = x1 + ks0 + jnp.uint32(5)
    return x0 ^ x1


def _sample_tile(p0, p1, bi, bj, n, blk):
    p = p1 / (p0 + p1)
    r = jax.lax.broadcasted_iota(jnp.uint32, (blk, blk), 0)
    c = jax.lax.broadcasted_iota(jnp.uint32, (blk, blk), 1)
    idx = (bi * blk + r) * jnp.uint32(n) + (bj * blk + c)
    bits = _threefry_bits(idx)
    u = jax.lax.bitcast_convert_type(
        (bits >> 9) | jnp.uint32(0x3F800000), jnp.float32
    ) - 1.0
    return (u < p).astype(jnp.int32)


def _sample_sym_kernel(steps_ref, p0_ref, p1_ref, out_ref, scratch_ref, *, n, blk):
    s = pl.program_id(0)
    bi = steps_ref[s, 0].astype(jnp.uint32)
    bj = steps_ref[s, 1].astype(jnp.uint32)
    mode = steps_ref[s, 4]

    @pl.when(mode == 0)
    def _():
        a = _sample_tile(p0_ref[...], p1_ref[...], bi, bj, n, blk)
        out_ref[...] = a
        scratch_ref[...] = a

    @pl.when(mode == 1)
    def _():
        out_ref[...] = scratch_ref[...].T

    @pl.when(mode == 2)
    def _():
        a = _sample_tile(p0_ref[...], p1_ref[...], bi, bj, n, blk)
        r = jax.lax.broadcasted_iota(jnp.int32, (blk, blk), 0)
        c = jax.lax.broadcasted_iota(jnp.int32, (blk, blk), 1)
        out_ref[...] = jnp.where(r <= c, a, a.T)


def _make_steps(nb):
    rows = []
    for i in range(nb):
        rows.append((i, i, i, i, 2))
        for j in range(i + 1, nb):
            rows.append((i, j, i, j, 0))
            rows.append((i, j, j, i, 1))
    return np.asarray(rows, dtype=np.int32)


@jax.jit
def kernel(prob_A):
    n = prob_A.shape[0]
    blk = _BLK
    nb = n // blk
    steps = jnp.asarray(_make_steps(nb))
    p0 = prob_A[..., 0]
    p1 = prob_A[..., 1]
    grid_spec = pltpu.PrefetchScalarGridSpec(
        num_scalar_prefetch=1,
        grid=(steps.shape[0],),
        in_specs=[
            pl.BlockSpec((blk, blk), lambda s, t: (t[s, 0], t[s, 1])),
            pl.BlockSpec((blk, blk), lambda s, t: (t[s, 0], t[s, 1])),
        ],
        out_specs=pl.BlockSpec((blk, blk), lambda s, t: (t[s, 2], t[s, 3])),
        scratch_shapes=[pltpu.VMEM((blk, blk), jnp.int32)],
    )
    return pl.pallas_call(
        functools.partial(_sample_sym_kernel, n=n, blk=blk),
        grid_spec=grid_spec,
        out_shape=jax.ShapeDtypeStruct((n, n), jnp.int32),
        compiler_params=pltpu.CompilerParams(
            dimension_semantics=("arbitrary",),
        ),
    )(steps, p0, p1)
